# baseline (speedup 1.0000x reference)
import functools

import jax
import jax.numpy as jnp
from jax import lax
from jax.experimental import pallas as pl
from jax.experimental.pallas import tpu as pltpu
from jax.experimental.pallas import tpu_sc as plsc

N_NODES = 10000
N_EDGES = 320000
FEATS = 128

C_ROWS = 10016
C_FLAT = C_ROWS * 16

BA = 8000
NBLK_A = N_EDGES // BA


def _sums_body(x_ref, dst_ref, w_ref, z_ref, sums_acc, cnt_acc):
    i = pl.program_id(0)

    @pl.when(i == 0)
    def _():
        sums_acc[...] = jnp.zeros((16, FEATS), jnp.float32)
        cnt_acc[...] = jnp.zeros((16, FEATS), jnp.float32)

    d = dst_ref[0]
    oh = (lax.broadcasted_iota(jnp.int32, (16, BA), 0) == d).astype(jnp.float32)
    sums_acc[...] += lax.dot_general(
        oh, x_ref[...], (((1,), (0,)), ((), ())),
        preferred_element_type=jnp.float32)
    cnt_acc[...] += jnp.broadcast_to(
        jnp.sum(oh, axis=1, keepdims=True), (16, FEATS))

    @pl.when(i == NBLK_A - 1)
    def _():
        nh = sums_acc[...] / jnp.maximum(cnt_acc[...], 1.0)
        row = lax.broadcasted_iota(jnp.int32, (16, FEATS), 0)
        nh = jnp.where(row < 14, nh, 0.0)
        z_ref[...] = 0.5 * lax.dot_general(
            nh, w_ref[...], (((1,), (1,)), ((), ())),
            preferred_element_type=jnp.float32)


_sums_call = pl.pallas_call(
    _sums_body,
    grid=(NBLK_A,),
    in_specs=[
        pl.BlockSpec((BA, FEATS), lambda i: (i, 0)),
        pl.BlockSpec((1, 1, BA), lambda i: (i, 0, 0)),
        pl.BlockSpec((FEATS, FEATS), lambda i: (0, 0)),
    ],
    out_specs=pl.BlockSpec((16, FEATS), lambda i: (0, 0)),
    out_shape=jax.ShapeDtypeStruct((16, FEATS), jnp.float32),
    scratch_shapes=[
        pltpu.VMEM((16, FEATS), jnp.float32),
        pltpu.VMEM((16, FEATS), jnp.float32),
    ],
)

EB = N_EDGES // 16
EB_PAD = 20096
NCH_B = EB_PAD // 128
SLICE_B = C_FLAT // 16


def _scatter_body(src_hbm, dst_hbm, c_hbm, src_v, dst_v, idx_v, ones_v,
                  wbuf_v, c_sp, sem):
    w = lax.axis_index("s")
    base = w * EB

    pltpu.async_copy(src_hbm.at[pl.ds(base, EB)], src_v.at[pl.ds(0, EB)],
                     sem).wait()
    pltpu.async_copy(dst_hbm.at[pl.ds(base, EB)], dst_v.at[pl.ds(0, EB)],
                     sem).wait()
    for k in range(6):
        src_v[pl.ds(EB + k * 16, 16)] = jnp.full((16,), 15, jnp.int32)
        dst_v[pl.ds(EB + k * 16, 16)] = jnp.full((16,), C_ROWS - 1, jnp.int32)

    for k in range(8):
        ones_v[pl.ds(k * 16, 16)] = jnp.ones((16,), jnp.float32)

    def zero_body(i, c):
        wbuf_v[pl.ds(i * 16, 16)] = jnp.zeros((16,), jnp.float32)
        return c
    lax.fori_loop(0, SLICE_B // 16, zero_body, 0)
    pltpu.sync_copy(wbuf_v, c_sp.at[pl.ds(w * SLICE_B, SLICE_B)])
    plsc.subcore_barrier()

    def idx_body(j, c):
        for k in range(8):
            p = j * 128 + k * 16
            s = src_v[pl.ds(p, 16)]
            d = dst_v[pl.ds(p, 16)]
            s = jnp.minimum(jnp.maximum(s, 0), 15)
            d = jnp.minimum(jnp.maximum(d, 0), C_ROWS - 1)
            idx_v[j, pl.ds(k * 16, 16)] = d * 16 + s
        return c
    lax.fori_loop(0, NCH_B, idx_body, 0)

    def sc_body(j, c):
        pltpu.sync_copy(ones_v, c_sp.at[idx_v.at[j]], add=True)
        return c
    lax.fori_loop(0, NCH_B, sc_body, 0)
    plsc.subcore_barrier()

    pltpu.sync_copy(c_sp.at[pl.ds(w * SLICE_B, SLICE_B)], wbuf_v)
    pltpu.sync_copy(wbuf_v, c_hbm.at[pl.ds(w * SLICE_B, SLICE_B)])


@functools.cache
def _scatter_call():
    return pl.kernel(
        _scatter_body,
        out_type=jax.ShapeDtypeStruct((C_FLAT,), jnp.float32),
        mesh=plsc.VectorSubcoreMesh(core_axis_name="c", subcore_axis_name="s",
                                    num_cores=1),
        scratch_types=[
            pltpu.VMEM((EB_PAD,), jnp.int32),
            pltpu.VMEM((EB_PAD,), jnp.int32),
            pltpu.VMEM((NCH_B, 128), jnp.int32),
            pltpu.VMEM((128,), jnp.float32),
            pltpu.VMEM((SLICE_B,), jnp.float32),
            pltpu.VMEM_SHARED((C_FLAT,), jnp.float32),
            pltpu.SemaphoreType.DMA,
        ],
    )


def _y_body(c_ref, z_ref, b_ref, y_ref):
    y_ref[...] = lax.dot_general(
        c_ref[...], z_ref[...], (((1,), (0,)), ((), ())),
        preferred_element_type=jnp.float32) + 0.5 * b_ref[...]


_y_call = pl.pallas_call(
    _y_body,
    grid=(1,),
    in_specs=[
        pl.BlockSpec((C_ROWS, 16), lambda i: (0, 0)),
        pl.BlockSpec((16, FEATS), lambda i: (0, 0)),
        pl.BlockSpec((1, FEATS), lambda i: (0, 0)),
    ],
    out_specs=pl.BlockSpec((C_ROWS, FEATS), lambda i: (0, 0)),
    out_shape=jax.ShapeDtypeStruct((C_ROWS, FEATS), jnp.float32),
)

EC = N_EDGES // 32
EC_PAD = 10112
NCH = EC_PAD // 128
REM_C = EC - (NCH - 1) * 128


def _gadd_body(y_hbm, src_hbm, dst_hbm, out_hbm, idx_s, idx_d,
               rs0, rd0, rs1, rd1, rs2, rd2, sem0, sem1, sem2):
    w = lax.axis_index("s") * 2 + lax.axis_index("c")
    base = w * EC
    rs = (rs0, rs1, rs2)
    rd = (rd0, rd1, rd2)
    sems = (sem0, sem1, sem2)

    pltpu.async_copy(src_hbm.at[pl.ds(base, EC)], idx_s.at[pl.ds(0, EC)],
                     sem0).wait()
    pltpu.async_copy(dst_hbm.at[pl.ds(base, EC)], idx_d.at[pl.ds(0, EC)],
                     sem0).wait()
    for k in range((EC_PAD - EC) // 16):
        idx_s[pl.ds(EC + k * 16, 16)] = jnp.zeros((16,), jnp.int32)
        idx_d[pl.ds(EC + k * 16, 16)] = jnp.zeros((16,), jnp.int32)

    def fire(c, p):
        pltpu.async_copy(y_hbm.at[idx_s.at[pl.ds(c * 128, 128)]], rs[p],
                         sems[p])
        pltpu.async_copy(y_hbm.at[idx_d.at[pl.ds(c * 128, 128)]], rd[p],
                         sems[p])

    def drain(p):
        pltpu.make_async_copy(y_hbm.at[pl.ds(0, 128)], rs[p], sems[p]).wait()
        pltpu.make_async_copy(y_hbm.at[pl.ds(0, 128)], rd[p], sems[p]).wait()

    def add_rows(p):
        def body(i, c):
            for k in range(FEATS // 16):
                sl = pl.ds(k * 16, 16)
                rs[p][i, sl] = rs[p][i, sl] + rd[p][i, sl]
            return c
        lax.fori_loop(0, 128, body, 0, unroll=2)

    fire(0, 0)
    fire(1, 1)

    def loop_body(t, carry):
        for i in range(3):
            c = 3 * t + i
            drain(i)
            add_rows(i)
            pltpu.sync_copy(rs[i], out_hbm.at[pl.ds(base + c * 128, 128)])

            @pl.when(c + 2 <= NCH - 1)
            def _():
                fire(c + 2, (i + 2) % 3)
        return carry
    lax.fori_loop(0, (NCH - 1) // 3, loop_body, 0)

    drain(0)
    add_rows(0)
    pltpu.sync_copy(rs[0].at[pl.ds(0, REM_C)],
                    out_hbm.at[pl.ds(base + (NCH - 1) * 128, REM_C)])


@functools.cache
def _gadd_call():
    return pl.kernel(
        _gadd_body,
        out_type=jax.ShapeDtypeStruct((N_EDGES, FEATS), jnp.float32),
        mesh=plsc.VectorSubcoreMesh(core_axis_name="c", subcore_axis_name="s"),
        scratch_types=[
            pltpu.VMEM((EC_PAD,), jnp.int32),
            pltpu.VMEM((EC_PAD,), jnp.int32),
            pltpu.VMEM((128, FEATS), jnp.float32),
            pltpu.VMEM((128, FEATS), jnp.float32),
            pltpu.VMEM((128, FEATS), jnp.float32),
            pltpu.VMEM((128, FEATS), jnp.float32),
            pltpu.VMEM((128, FEATS), jnp.float32),
            pltpu.VMEM((128, FEATS), jnp.float32),
            pltpu.SemaphoreType.DMA,
            pltpu.SemaphoreType.DMA,
            pltpu.SemaphoreType.DMA,
        ],
    )


def kernel(inputs, edge_index, W, b):
    src = edge_index[0]
    dst = edge_index[1]
    dst3d = dst.reshape(NBLK_A, 1, BA)
    z = _sums_call(inputs, dst3d, W)
    c_flat = _scatter_call()(src, dst)
    y = _y_call(c_flat.reshape(C_ROWS, 16), z, b.reshape(1, FEATS))
    return _gadd_call()(y, src, dst)


# speedup vs baseline: 1.4168x; 1.4168x over previous
import functools

import jax
import jax.numpy as jnp
from jax import lax
from jax.experimental import pallas as pl
from jax.experimental.pallas import tpu as pltpu
from jax.experimental.pallas import tpu_sc as plsc

N_NODES = 10000
N_EDGES = 320000
FEATS = 128

C_ROWS = 10016
C_FLAT = C_ROWS * 16

BA = 8000
NBLK_A = N_EDGES // BA


def _sums_body(x_ref, dst_ref, w_ref, z_ref, sums_acc, cnt_acc):
    i = pl.program_id(0)

    @pl.when(i == 0)
    def _():
        sums_acc[...] = jnp.zeros((16, FEATS), jnp.float32)
        cnt_acc[...] = jnp.zeros((16, FEATS), jnp.float32)

    d = dst_ref[0]
    oh = (lax.broadcasted_iota(jnp.int32, (16, BA), 0) == d).astype(jnp.float32)
    sums_acc[...] += lax.dot_general(
        oh, x_ref[...], (((1,), (0,)), ((), ())),
        preferred_element_type=jnp.float32)
    cnt_acc[...] += jnp.broadcast_to(
        jnp.sum(oh, axis=1, keepdims=True), (16, FEATS))

    @pl.when(i == NBLK_A - 1)
    def _():
        nh = sums_acc[...] / jnp.maximum(cnt_acc[...], 1.0)
        row = lax.broadcasted_iota(jnp.int32, (16, FEATS), 0)
        nh = jnp.where(row < 14, nh, 0.0)
        z_ref[...] = 0.5 * lax.dot_general(
            nh, w_ref[...], (((1,), (1,)), ((), ())),
            preferred_element_type=jnp.float32)


_sums_call = pl.pallas_call(
    _sums_body,
    grid=(NBLK_A,),
    in_specs=[
        pl.BlockSpec((BA, FEATS), lambda i: (i, 0)),
        pl.BlockSpec((1, 1, BA), lambda i: (i, 0, 0)),
        pl.BlockSpec((FEATS, FEATS), lambda i: (0, 0)),
    ],
    out_specs=pl.BlockSpec((16, FEATS), lambda i: (0, 0)),
    out_shape=jax.ShapeDtypeStruct((16, FEATS), jnp.float32),
    scratch_shapes=[
        pltpu.VMEM((16, FEATS), jnp.float32),
        pltpu.VMEM((16, FEATS), jnp.float32),
    ],
)

EB = N_EDGES // 16
EB_PAD = 20096
NCH_B = EB_PAD // 128
SLICE_B = C_FLAT // 16


def _scatter_body(src_hbm, dst_hbm, c_hbm, src_v, dst_v, idx_v, ones_v,
                  wbuf_v, c_sp, sem):
    w = lax.axis_index("s")
    base = w * EB

    pltpu.async_copy(src_hbm.at[pl.ds(base, EB)], src_v.at[pl.ds(0, EB)],
                     sem).wait()
    pltpu.async_copy(dst_hbm.at[pl.ds(base, EB)], dst_v.at[pl.ds(0, EB)],
                     sem).wait()
    for k in range(6):
        src_v[pl.ds(EB + k * 16, 16)] = jnp.full((16,), 15, jnp.int32)
        dst_v[pl.ds(EB + k * 16, 16)] = jnp.full((16,), C_ROWS - 1, jnp.int32)

    for k in range(8):
        ones_v[pl.ds(k * 16, 16)] = jnp.ones((16,), jnp.float32)

    def zero_body(i, c):
        wbuf_v[pl.ds(i * 16, 16)] = jnp.zeros((16,), jnp.float32)
        return c
    lax.fori_loop(0, SLICE_B // 16, zero_body, 0)
    pltpu.sync_copy(wbuf_v, c_sp.at[pl.ds(w * SLICE_B, SLICE_B)])
    plsc.subcore_barrier()

    def idx_body(j, c):
        for k in range(8):
            p = j * 128 + k * 16
            s = src_v[pl.ds(p, 16)]
            d = dst_v[pl.ds(p, 16)]
            s = jnp.minimum(jnp.maximum(s, 0), 15)
            d = jnp.minimum(jnp.maximum(d, 0), C_ROWS - 1)
            idx_v[j, pl.ds(k * 16, 16)] = d * 16 + s
        return c
    lax.fori_loop(0, NCH_B, idx_body, 0)

    def sc_body(j, c):
        pltpu.sync_copy(ones_v, c_sp.at[idx_v.at[j]], add=True)
        return c
    lax.fori_loop(0, NCH_B, sc_body, 0)
    plsc.subcore_barrier()

    pltpu.sync_copy(c_sp.at[pl.ds(w * SLICE_B, SLICE_B)], wbuf_v)
    pltpu.sync_copy(wbuf_v, c_hbm.at[pl.ds(w * SLICE_B, SLICE_B)])


@functools.cache
def _scatter_call():
    return pl.kernel(
        _scatter_body,
        out_type=jax.ShapeDtypeStruct((C_FLAT,), jnp.float32),
        mesh=plsc.VectorSubcoreMesh(core_axis_name="c", subcore_axis_name="s",
                                    num_cores=1),
        scratch_types=[
            pltpu.VMEM((EB_PAD,), jnp.int32),
            pltpu.VMEM((EB_PAD,), jnp.int32),
            pltpu.VMEM((NCH_B, 128), jnp.int32),
            pltpu.VMEM((128,), jnp.float32),
            pltpu.VMEM((SLICE_B,), jnp.float32),
            pltpu.VMEM_SHARED((C_FLAT,), jnp.float32),
            pltpu.SemaphoreType.DMA,
        ],
    )


def _y_body(c_ref, z_ref, b_ref, y_ref):
    y_ref[...] = lax.dot_general(
        c_ref[...], z_ref[...], (((1,), (0,)), ((), ())),
        preferred_element_type=jnp.float32) + 0.5 * b_ref[...]


_y_call = pl.pallas_call(
    _y_body,
    grid=(1,),
    in_specs=[
        pl.BlockSpec((C_ROWS, 16), lambda i: (0, 0)),
        pl.BlockSpec((16, FEATS), lambda i: (0, 0)),
        pl.BlockSpec((1, FEATS), lambda i: (0, 0)),
    ],
    out_specs=pl.BlockSpec((C_ROWS, FEATS), lambda i: (0, 0)),
    out_shape=jax.ShapeDtypeStruct((C_ROWS, FEATS), jnp.float32),
)

EC = N_EDGES // 32
EC_PAD = 10112
NCH = EC_PAD // 128
REM_C = EC - (NCH - 1) * 128


def _gadd_body(y_hbm, src_hbm, dst_hbm, out_hbm, idx_s, idx_d,
               rs0, rd0, rs1, rd1, rs2, rd2, sem0, sem1, sem2):
    w = lax.axis_index("s") * 2 + lax.axis_index("c")
    base = w * EC
    rs = (rs0, rs1, rs2)
    rd = (rd0, rd1, rd2)
    sems = (sem0, sem1, sem2)

    pltpu.async_copy(src_hbm.at[pl.ds(base, EC)], idx_s.at[pl.ds(0, EC)],
                     sem0).wait()
    pltpu.async_copy(dst_hbm.at[pl.ds(base, EC)], idx_d.at[pl.ds(0, EC)],
                     sem0).wait()
    for k in range((EC_PAD - EC) // 16):
        idx_s[pl.ds(EC + k * 16, 16)] = jnp.zeros((16,), jnp.int32)
        idx_d[pl.ds(EC + k * 16, 16)] = jnp.zeros((16,), jnp.int32)

    def fire(c, p):
        pltpu.async_copy(y_hbm.at[idx_s.at[pl.ds(c * 128, 128)]], rs[p],
                         sems[p])
        pltpu.async_copy(y_hbm.at[idx_d.at[pl.ds(c * 128, 128)]], rd[p],
                         sems[p])

    def drain(p):
        pltpu.make_async_copy(y_hbm.at[pl.ds(0, 128)], rs[p], sems[p]).wait()
        pltpu.make_async_copy(y_hbm.at[pl.ds(0, 128)], rd[p], sems[p]).wait()

    def add_rows(p):
        @plsc.parallel_loop(0, 128, 1, unroll=4)
        def _(i):
            for k in range(FEATS // 16):
                sl = pl.ds(k * 16, 16)
                rs[p][i, sl] = rs[p][i, sl] + rd[p][i, sl]

    fire(0, 0)
    fire(1, 1)

    def loop_body(t, carry):
        for i in range(3):
            c = 3 * t + i
            drain(i)
            add_rows(i)
            pltpu.sync_copy(rs[i], out_hbm.at[pl.ds(base + c * 128, 128)])

            @pl.when(c + 2 <= NCH - 1)
            def _():
                fire(c + 2, (i + 2) % 3)
        return carry
    lax.fori_loop(0, (NCH - 1) // 3, loop_body, 0)

    drain(0)
    add_rows(0)
    pltpu.sync_copy(rs[0].at[pl.ds(0, REM_C)],
                    out_hbm.at[pl.ds(base + (NCH - 1) * 128, REM_C)])


@functools.cache
def _gadd_call():
    return pl.kernel(
        _gadd_body,
        out_type=jax.ShapeDtypeStruct((N_EDGES, FEATS), jnp.float32),
        mesh=plsc.VectorSubcoreMesh(core_axis_name="c", subcore_axis_name="s"),
        scratch_types=[
            pltpu.VMEM((EC_PAD,), jnp.int32),
            pltpu.VMEM((EC_PAD,), jnp.int32),
            pltpu.VMEM((128, FEATS), jnp.float32),
            pltpu.VMEM((128, FEATS), jnp.float32),
            pltpu.VMEM((128, FEATS), jnp.float32),
            pltpu.VMEM((128, FEATS), jnp.float32),
            pltpu.VMEM((128, FEATS), jnp.float32),
            pltpu.VMEM((128, FEATS), jnp.float32),
            pltpu.SemaphoreType.DMA,
            pltpu.SemaphoreType.DMA,
            pltpu.SemaphoreType.DMA,
        ],
    )


def kernel(inputs, edge_index, W, b):
    src = edge_index[0]
    dst = edge_index[1]
    dst3d = dst.reshape(NBLK_A, 1, BA)
    z = _sums_call(inputs, dst3d, W)
    c_flat = _scatter_call()(src, dst)
    y = _y_call(c_flat.reshape(C_ROWS, 16), z, b.reshape(1, FEATS))
    return _gadd_call()(y, src, dst)


# speedup vs baseline: 1.4269x; 1.0071x over previous
import functools

import jax
import jax.numpy as jnp
from jax import lax
from jax.experimental import pallas as pl
from jax.experimental.pallas import tpu as pltpu
from jax.experimental.pallas import tpu_sc as plsc

N_NODES = 10000
N_EDGES = 320000
FEATS = 128

C_ROWS = 10016
C_FLAT = C_ROWS * 16

BA = 8000
NBLK_A = N_EDGES // BA


def _sums_body(x_ref, dst_ref, w_ref, z_ref, sums_acc, cnt_acc):
    i = pl.program_id(0)

    @pl.when(i == 0)
    def _():
        sums_acc[...] = jnp.zeros((16, FEATS), jnp.float32)
        cnt_acc[...] = jnp.zeros((16, FEATS), jnp.float32)

    d = dst_ref[0]
    oh = (lax.broadcasted_iota(jnp.int32, (16, BA), 0) == d).astype(jnp.float32)
    sums_acc[...] += lax.dot_general(
        oh, x_ref[...], (((1,), (0,)), ((), ())),
        preferred_element_type=jnp.float32)
    cnt_acc[...] += jnp.broadcast_to(
        jnp.sum(oh, axis=1, keepdims=True), (16, FEATS))

    @pl.when(i == NBLK_A - 1)
    def _():
        nh = sums_acc[...] / jnp.maximum(cnt_acc[...], 1.0)
        row = lax.broadcasted_iota(jnp.int32, (16, FEATS), 0)
        nh = jnp.where(row < 14, nh, 0.0)
        z_ref[...] = 0.5 * lax.dot_general(
            nh, w_ref[...], (((1,), (1,)), ((), ())),
            preferred_element_type=jnp.float32)


_sums_call = pl.pallas_call(
    _sums_body,
    grid=(NBLK_A,),
    in_specs=[
        pl.BlockSpec((BA, FEATS), lambda i: (i, 0)),
        pl.BlockSpec((1, 1, BA), lambda i: (i, 0, 0)),
        pl.BlockSpec((FEATS, FEATS), lambda i: (0, 0)),
    ],
    out_specs=pl.BlockSpec((16, FEATS), lambda i: (0, 0)),
    out_shape=jax.ShapeDtypeStruct((16, FEATS), jnp.float32),
    scratch_shapes=[
        pltpu.VMEM((16, FEATS), jnp.float32),
        pltpu.VMEM((16, FEATS), jnp.float32),
    ],
)

EB = N_EDGES // 16
EB_PAD = 20096
NCH_B = EB_PAD // 128
SLICE_B = C_FLAT // 16


def _scatter_body(src_hbm, dst_hbm, c_hbm, src_v, dst_v, idx_v, ones_v,
                  wbuf_v, c_sp, sem):
    w = lax.axis_index("s")
    base = w * EB

    pltpu.async_copy(src_hbm.at[pl.ds(base, EB)], src_v.at[pl.ds(0, EB)],
                     sem).wait()
    pltpu.async_copy(dst_hbm.at[pl.ds(base, EB)], dst_v.at[pl.ds(0, EB)],
                     sem).wait()
    for k in range(6):
        src_v[pl.ds(EB + k * 16, 16)] = jnp.full((16,), 15, jnp.int32)
        dst_v[pl.ds(EB + k * 16, 16)] = jnp.full((16,), C_ROWS - 1, jnp.int32)

    for k in range(8):
        ones_v[pl.ds(k * 16, 16)] = jnp.ones((16,), jnp.float32)

    def zero_body(i, c):
        wbuf_v[pl.ds(i * 16, 16)] = jnp.zeros((16,), jnp.float32)
        return c
    lax.fori_loop(0, SLICE_B // 16, zero_body, 0)
    pltpu.sync_copy(wbuf_v, c_sp.at[pl.ds(w * SLICE_B, SLICE_B)])
    plsc.subcore_barrier()

    def idx_body(j, c):
        for k in range(8):
            p = j * 128 + k * 16
            s = src_v[pl.ds(p, 16)]
            d = dst_v[pl.ds(p, 16)]
            s = jnp.minimum(jnp.maximum(s, 0), 15)
            d = jnp.minimum(jnp.maximum(d, 0), C_ROWS - 1)
            idx_v[j, pl.ds(k * 16, 16)] = d * 16 + s
        return c
    lax.fori_loop(0, NCH_B, idx_body, 0)

    def sc_body(j, c):
        pltpu.sync_copy(ones_v, c_sp.at[idx_v.at[j]], add=True)
        return c
    lax.fori_loop(0, NCH_B, sc_body, 0)
    plsc.subcore_barrier()

    pltpu.sync_copy(c_sp.at[pl.ds(w * SLICE_B, SLICE_B)], wbuf_v)
    pltpu.sync_copy(wbuf_v, c_hbm.at[pl.ds(w * SLICE_B, SLICE_B)])


@functools.cache
def _scatter_call():
    return pl.kernel(
        _scatter_body,
        out_type=jax.ShapeDtypeStruct((C_FLAT,), jnp.float32),
        mesh=plsc.VectorSubcoreMesh(core_axis_name="c", subcore_axis_name="s",
                                    num_cores=1),
        scratch_types=[
            pltpu.VMEM((EB_PAD,), jnp.int32),
            pltpu.VMEM((EB_PAD,), jnp.int32),
            pltpu.VMEM((NCH_B, 128), jnp.int32),
            pltpu.VMEM((128,), jnp.float32),
            pltpu.VMEM((SLICE_B,), jnp.float32),
            pltpu.VMEM_SHARED((C_FLAT,), jnp.float32),
            pltpu.SemaphoreType.DMA,
        ],
    )


def _y_body(c_ref, z_ref, b_ref, y_ref):
    y_ref[...] = lax.dot_general(
        c_ref[...], z_ref[...], (((1,), (0,)), ((), ())),
        preferred_element_type=jnp.float32) + 0.5 * b_ref[...]


_y_call = pl.pallas_call(
    _y_body,
    grid=(1,),
    in_specs=[
        pl.BlockSpec((C_ROWS, 16), lambda i: (0, 0)),
        pl.BlockSpec((16, FEATS), lambda i: (0, 0)),
        pl.BlockSpec((1, FEATS), lambda i: (0, 0)),
    ],
    out_specs=pl.BlockSpec((C_ROWS, FEATS), lambda i: (0, 0)),
    out_shape=jax.ShapeDtypeStruct((C_ROWS, FEATS), jnp.float32),
)

EC = N_EDGES // 32
EC_PAD = 10112
NCH = EC_PAD // 128
REM_C = EC - (NCH - 1) * 128


def _gadd_body(y_hbm, src_hbm, dst_hbm, out_hbm, idx_s, idx_d,
               rs0, rd0, rs1, rd1, rs2, rd2, sem0, sem1, sem2,
               wsem0, wsem1, wsem2):
    w = lax.axis_index("s") * 2 + lax.axis_index("c")
    base = w * EC
    rs = (rs0, rs1, rs2)
    rd = (rd0, rd1, rd2)
    sems = (sem0, sem1, sem2)
    wsems = (wsem0, wsem1, wsem2)

    pltpu.async_copy(src_hbm.at[pl.ds(base, EC)], idx_s.at[pl.ds(0, EC)],
                     sem0).wait()
    pltpu.async_copy(dst_hbm.at[pl.ds(base, EC)], idx_d.at[pl.ds(0, EC)],
                     sem0).wait()
    for k in range((EC_PAD - EC) // 16):
        idx_s[pl.ds(EC + k * 16, 16)] = jnp.zeros((16,), jnp.int32)
        idx_d[pl.ds(EC + k * 16, 16)] = jnp.zeros((16,), jnp.int32)

    def fire(c, p):
        pltpu.async_copy(y_hbm.at[idx_s.at[pl.ds(c * 128, 128)]], rs[p],
                         sems[p])
        pltpu.async_copy(y_hbm.at[idx_d.at[pl.ds(c * 128, 128)]], rd[p],
                         sems[p])

    def drain(p):
        pltpu.make_async_copy(y_hbm.at[pl.ds(0, 128)], rs[p], sems[p]).wait()
        pltpu.make_async_copy(y_hbm.at[pl.ds(0, 128)], rd[p], sems[p]).wait()

    def add_rows(p):
        @plsc.parallel_loop(0, 128, 1, unroll=4)
        def _(i):
            for k in range(FEATS // 16):
                sl = pl.ds(k * 16, 16)
                rs[p][i, sl] = rs[p][i, sl] + rd[p][i, sl]

    fire(0, 0)
    fire(1, 1)

    def wdrain(p):
        pltpu.make_async_copy(rs[p], out_hbm.at[pl.ds(base, 128)],
                              wsems[p]).wait()

    def loop_body(t, carry):
        for i in range(3):
            c = 3 * t + i
            drain(i)
            add_rows(i)
            pltpu.async_copy(rs[i], out_hbm.at[pl.ds(base + c * 128, 128)],
                             wsems[i])

            @pl.when(c + 2 <= NCH - 1)
            def _():
                @pl.when(c >= 1)
                def _():
                    wdrain((i + 2) % 3)
                fire(c + 2, (i + 2) % 3)
        return carry
    lax.fori_loop(0, (NCH - 1) // 3, loop_body, 0)

    wdrain(1)
    wdrain(2)

    drain(0)
    add_rows(0)
    pltpu.sync_copy(rs[0].at[pl.ds(0, REM_C)],
                    out_hbm.at[pl.ds(base + (NCH - 1) * 128, REM_C)])


@functools.cache
def _gadd_call():
    return pl.kernel(
        _gadd_body,
        out_type=jax.ShapeDtypeStruct((N_EDGES, FEATS), jnp.float32),
        mesh=plsc.VectorSubcoreMesh(core_axis_name="c", subcore_axis_name="s"),
        scratch_types=[
            pltpu.VMEM((EC_PAD,), jnp.int32),
            pltpu.VMEM((EC_PAD,), jnp.int32),
            pltpu.VMEM((128, FEATS), jnp.float32),
            pltpu.VMEM((128, FEATS), jnp.float32),
            pltpu.VMEM((128, FEATS), jnp.float32),
            pltpu.VMEM((128, FEATS), jnp.float32),
            pltpu.VMEM((128, FEATS), jnp.float32),
            pltpu.VMEM((128, FEATS), jnp.float32),
            pltpu.SemaphoreType.DMA,
            pltpu.SemaphoreType.DMA,
            pltpu.SemaphoreType.DMA,
            pltpu.SemaphoreType.DMA,
            pltpu.SemaphoreType.DMA,
            pltpu.SemaphoreType.DMA,
        ],
    )


def kernel(inputs, edge_index, W, b):
    src = edge_index[0]
    dst = edge_index[1]
    dst3d = dst.reshape(NBLK_A, 1, BA)
    z = _sums_call(inputs, dst3d, W)
    c_flat = _scatter_call()(src, dst)
    y = _y_call(c_flat.reshape(C_ROWS, 16), z, b.reshape(1, FEATS))
    return _gadd_call()(y, src, dst)


# speedup vs baseline: 1.4279x; 1.0007x over previous
import functools

import jax
import jax.numpy as jnp
from jax import lax
from jax.experimental import pallas as pl
from jax.experimental.pallas import tpu as pltpu
from jax.experimental.pallas import tpu_sc as plsc

N_NODES = 10000
N_EDGES = 320000
FEATS = 128

C_ROWS = 10016
C_FLAT = C_ROWS * 16

BA = 8000
NBLK_A = N_EDGES // BA


def _sums_body(x_ref, dst_ref, w_ref, z_ref, sums_acc, cnt_acc):
    i = pl.program_id(0)

    @pl.when(i == 0)
    def _():
        sums_acc[...] = jnp.zeros((16, FEATS), jnp.float32)
        cnt_acc[...] = jnp.zeros((16, FEATS), jnp.float32)

    d = dst_ref[0]
    oh = (lax.broadcasted_iota(jnp.int32, (16, BA), 0) == d).astype(jnp.float32)
    sums_acc[...] += lax.dot_general(
        oh, x_ref[...], (((1,), (0,)), ((), ())),
        preferred_element_type=jnp.float32)
    cnt_acc[...] += jnp.broadcast_to(
        jnp.sum(oh, axis=1, keepdims=True), (16, FEATS))

    @pl.when(i == NBLK_A - 1)
    def _():
        nh = sums_acc[...] / jnp.maximum(cnt_acc[...], 1.0)
        row = lax.broadcasted_iota(jnp.int32, (16, FEATS), 0)
        nh = jnp.where(row < 14, nh, 0.0)
        z_ref[...] = 0.5 * lax.dot_general(
            nh, w_ref[...], (((1,), (1,)), ((), ())),
            preferred_element_type=jnp.float32)


_sums_call = pl.pallas_call(
    _sums_body,
    grid=(NBLK_A,),
    in_specs=[
        pl.BlockSpec((BA, FEATS), lambda i: (i, 0)),
        pl.BlockSpec((1, 1, BA), lambda i: (i, 0, 0)),
        pl.BlockSpec((FEATS, FEATS), lambda i: (0, 0)),
    ],
    out_specs=pl.BlockSpec((16, FEATS), lambda i: (0, 0)),
    out_shape=jax.ShapeDtypeStruct((16, FEATS), jnp.float32),
    scratch_shapes=[
        pltpu.VMEM((16, FEATS), jnp.float32),
        pltpu.VMEM((16, FEATS), jnp.float32),
    ],
)

EB = N_EDGES // 16
EB_PAD = 20096
NCH_B = EB_PAD // 128
SLICE_B = C_FLAT // 16


def _scatter_body(src_hbm, dst_hbm, c_hbm, src_v, dst_v, idx_v, ones_v,
                  wbuf_v, c_sp, sem):
    w = lax.axis_index("s")
    base = w * EB

    pltpu.async_copy(src_hbm.at[pl.ds(base, EB)], src_v.at[pl.ds(0, EB)],
                     sem).wait()
    pltpu.async_copy(dst_hbm.at[pl.ds(base, EB)], dst_v.at[pl.ds(0, EB)],
                     sem).wait()
    for k in range(6):
        src_v[pl.ds(EB + k * 16, 16)] = jnp.full((16,), 15, jnp.int32)
        dst_v[pl.ds(EB + k * 16, 16)] = jnp.full((16,), C_ROWS - 1, jnp.int32)

    for k in range(8):
        ones_v[pl.ds(k * 16, 16)] = jnp.ones((16,), jnp.float32)

    def zero_body(i, c):
        wbuf_v[pl.ds(i * 16, 16)] = jnp.zeros((16,), jnp.float32)
        return c
    lax.fori_loop(0, SLICE_B // 16, zero_body, 0)
    pltpu.sync_copy(wbuf_v, c_sp.at[pl.ds(w * SLICE_B, SLICE_B)])
    plsc.subcore_barrier()

    def idx_body(j, c):
        for k in range(8):
            p = j * 128 + k * 16
            s = src_v[pl.ds(p, 16)]
            d = dst_v[pl.ds(p, 16)]
            s = jnp.minimum(jnp.maximum(s, 0), 15)
            d = jnp.minimum(jnp.maximum(d, 0), C_ROWS - 1)
            idx_v[j, pl.ds(k * 16, 16)] = d * 16 + s
        return c
    lax.fori_loop(0, NCH_B, idx_body, 0)

    def sc_body(j, c):
        pltpu.sync_copy(ones_v, c_sp.at[idx_v.at[j]], add=True)
        return c
    lax.fori_loop(0, NCH_B, sc_body, 0)
    plsc.subcore_barrier()

    pltpu.sync_copy(c_sp.at[pl.ds(w * SLICE_B, SLICE_B)], wbuf_v)
    pltpu.sync_copy(wbuf_v, c_hbm.at[pl.ds(w * SLICE_B, SLICE_B)])


@functools.cache
def _scatter_call():
    return pl.kernel(
        _scatter_body,
        out_type=jax.ShapeDtypeStruct((C_FLAT,), jnp.float32),
        mesh=plsc.VectorSubcoreMesh(core_axis_name="c", subcore_axis_name="s",
                                    num_cores=1),
        scratch_types=[
            pltpu.VMEM((EB_PAD,), jnp.int32),
            pltpu.VMEM((EB_PAD,), jnp.int32),
            pltpu.VMEM((NCH_B, 128), jnp.int32),
            pltpu.VMEM((128,), jnp.float32),
            pltpu.VMEM((SLICE_B,), jnp.float32),
            pltpu.VMEM_SHARED((C_FLAT,), jnp.float32),
            pltpu.SemaphoreType.DMA,
        ],
    )


def _y_body(c_ref, z_ref, b_ref, y_ref):
    y_ref[...] = lax.dot_general(
        c_ref[...], z_ref[...], (((1,), (0,)), ((), ())),
        preferred_element_type=jnp.float32) + 0.5 * b_ref[...]


_y_call = pl.pallas_call(
    _y_body,
    grid=(1,),
    in_specs=[
        pl.BlockSpec((C_ROWS, 16), lambda i: (0, 0)),
        pl.BlockSpec((16, FEATS), lambda i: (0, 0)),
        pl.BlockSpec((1, FEATS), lambda i: (0, 0)),
    ],
    out_specs=pl.BlockSpec((C_ROWS, FEATS), lambda i: (0, 0)),
    out_shape=jax.ShapeDtypeStruct((C_ROWS, FEATS), jnp.float32),
)

EC = N_EDGES // 32
EC_PAD = 10112
NCH = EC_PAD // 128
REM_C = EC - (NCH - 1) * 128


def _gadd_body(y_hbm, src_hbm, dst_hbm, out_hbm, idx_s, idx_d,
               rs0, rd0, rs1, rd1, rs2, rd2, sem0, sem1, sem2,
               wsem0, wsem1, wsem2):
    w = lax.axis_index("s") * 2 + lax.axis_index("c")
    base = w * EC
    rs = (rs0, rs1, rs2)
    rd = (rd0, rd1, rd2)
    sems = (sem0, sem1, sem2)
    wsems = (wsem0, wsem1, wsem2)

    pltpu.async_copy(src_hbm.at[pl.ds(base, EC)], idx_s.at[pl.ds(0, EC)],
                     sem0).wait()
    pltpu.async_copy(dst_hbm.at[pl.ds(base, EC)], idx_d.at[pl.ds(0, EC)],
                     sem0).wait()
    for k in range((EC_PAD - EC) // 16):
        idx_s[pl.ds(EC + k * 16, 16)] = jnp.zeros((16,), jnp.int32)
        idx_d[pl.ds(EC + k * 16, 16)] = jnp.zeros((16,), jnp.int32)

    def fire(c, p):
        pltpu.async_copy(y_hbm.at[idx_s.at[pl.ds(c * 128, 128)]], rs[p],
                         sems[p])
        pltpu.async_copy(y_hbm.at[idx_d.at[pl.ds(c * 128, 128)]], rd[p],
                         sems[p])

    def drain(p):
        pltpu.make_async_copy(y_hbm.at[pl.ds(0, 128)], rs[p], sems[p]).wait()
        pltpu.make_async_copy(y_hbm.at[pl.ds(0, 128)], rd[p], sems[p]).wait()

    def add_rows(p):
        @plsc.parallel_loop(0, 128, 1, unroll=4)
        def _(i):
            for k in range(FEATS // 16):
                sl = pl.ds(k * 16, 16)
                rs[p][i, sl] = rs[p][i, sl] + rd[p][i, sl]

    fire(0, 0)
    fire(1, 1)

    def wdrain(p):
        pltpu.make_async_copy(rs[p], out_hbm.at[pl.ds(base, 128)],
                              wsems[p]).wait()

    def loop_body(t, carry):
        for i in range(3):
            c = 3 * t + i
            drain(i)

            @pl.when(c + 2 <= NCH - 1)
            def _():
                @pl.when(c >= 1)
                def _():
                    wdrain((i + 2) % 3)
                fire(c + 2, (i + 2) % 3)
            add_rows(i)
            pltpu.async_copy(rs[i], out_hbm.at[pl.ds(base + c * 128, 128)],
                             wsems[i])
        return carry
    lax.fori_loop(0, (NCH - 1) // 3, loop_body, 0)

    wdrain(1)
    wdrain(2)

    drain(0)
    add_rows(0)
    pltpu.sync_copy(rs[0].at[pl.ds(0, REM_C)],
                    out_hbm.at[pl.ds(base + (NCH - 1) * 128, REM_C)])


@functools.cache
def _gadd_call():
    return pl.kernel(
        _gadd_body,
        out_type=jax.ShapeDtypeStruct((N_EDGES, FEATS), jnp.float32),
        mesh=plsc.VectorSubcoreMesh(core_axis_name="c", subcore_axis_name="s"),
        scratch_types=[
            pltpu.VMEM((EC_PAD,), jnp.int32),
            pltpu.VMEM((EC_PAD,), jnp.int32),
            pltpu.VMEM((128, FEATS), jnp.float32),
            pltpu.VMEM((128, FEATS), jnp.float32),
            pltpu.VMEM((128, FEATS), jnp.float32),
            pltpu.VMEM((128, FEATS), jnp.float32),
            pltpu.VMEM((128, FEATS), jnp.float32),
            pltpu.VMEM((128, FEATS), jnp.float32),
            pltpu.SemaphoreType.DMA,
            pltpu.SemaphoreType.DMA,
            pltpu.SemaphoreType.DMA,
            pltpu.SemaphoreType.DMA,
            pltpu.SemaphoreType.DMA,
            pltpu.SemaphoreType.DMA,
        ],
    )


def kernel(inputs, edge_index, W, b):
    src = edge_index[0]
    dst = edge_index[1]
    dst3d = dst.reshape(NBLK_A, 1, BA)
    z = _sums_call(inputs, dst3d, W)
    c_flat = _scatter_call()(src, dst)
    y = _y_call(c_flat.reshape(C_ROWS, 16), z, b.reshape(1, FEATS))
    return _gadd_call()(y, src, dst)
